# baseline (speedup 1.0000x reference)
import functools

import jax
import jax.numpy as jnp
from jax import lax
from jax.experimental import pallas as pl
from jax.experimental.pallas import tpu as pltpu
from jax.experimental.pallas import tpu_sc as plsc

N = 50000
E = 800000
H = 64
T = 16
NG = 128
NL = 5
Q = 16

RPT_ALL = 200
EPR = 32 * RPT_ALL
EP = EPR * 128
RPT_SC = EPR // 16
NP_ = 50176
RPN = NP_ // 16
ZR = 112
RB = 512
GRID = NP_ // RB
EW = 1280
EPW = EP // EW
WPT = EPW // 16
NPCH = WPT // 2
WPT_ALL = EPW // 32

_mesh = plsc.VectorSubcoreMesh(core_axis_name="c", subcore_axis_name="s")


def _zeros16():
    return jnp.zeros((16,), jnp.float32)


@functools.partial(
    pl.kernel,
    mesh=_mesh,
    out_type=[jax.ShapeDtypeStruct((NP_, Q), jnp.float32),
              jax.ShapeDtypeStruct((NP_, Q), jnp.float32)],
    scratch_types=[
        pltpu.VMEM((WPT_ALL, EW), jnp.int32),
        pltpu.VMEM((EW, Q), jnp.float32),
        pltpu.VMEM((ZR, Q), jnp.float32),
        pltpu.VMEM_SHARED((NP_, Q), jnp.float32),
        pltpu.SemaphoreType.DMA,
    ],
    compiler_params=pltpu.CompilerParams(use_tc_tiling_on_sc=False),
)
def _deg(dst2d, ones2d, out0, out1, dstbuf, onesbuf, zbuf, acc, sem):
    cc = lax.axis_index("c")
    ss = lax.axis_index("s")
    wid = ss * 2 + cc

    pltpu.sync_copy(dst2d.at[pl.ds(wid * WPT_ALL, WPT_ALL)], dstbuf)
    pltpu.sync_copy(ones2d, onesbuf)

    def zrow(i, _):
        zbuf[i, :] = _zeros16()
        return 0
    lax.fori_loop(0, ZR, zrow, 0)

    def zacc(q, _):
        pltpu.sync_copy(zbuf, acc.at[pl.ds(ss * RPN + q * ZR, ZR)])
        return 0
    lax.fori_loop(0, RPN // ZR, zacc, 0)
    plsc.subcore_barrier()

    def body(k, _):
        pltpu.async_copy(onesbuf, acc.at[dstbuf.at[k]], sem, add=True)
        @pl.when(k >= 2)
        def _():
            pltpu.make_async_copy(onesbuf, acc.at[dstbuf.at[0]],
                                  sem).wait()
        return 0
    lax.fori_loop(0, WPT_ALL, body, 0)
    for b in range(2):
        pltpu.make_async_copy(onesbuf, acc.at[dstbuf.at[0]], sem).wait()
    plsc.subcore_barrier()

    @pl.when(cc == 0)
    def _():
        pltpu.sync_copy(acc.at[pl.ds(ss * RPN, RPN)],
                        out0.at[pl.ds(ss * RPN, RPN)])

    @pl.when(cc == 1)
    def _():
        pltpu.sync_copy(acc.at[pl.ds(ss * RPN, RPN)],
                        out1.at[pl.ds(ss * RPN, RPN)])


@functools.partial(
    pl.kernel,
    mesh=_mesh,
    out_type=jax.ShapeDtypeStruct((2 * NP_, Q), jnp.float32),
    scratch_types=[
        pltpu.VMEM((2, EW), jnp.int32),
        pltpu.VMEM((2, EW), jnp.int32),
        pltpu.VMEM((2, EW), jnp.int32),
        pltpu.VMEM((2, EW), jnp.int32),
        pltpu.VMEM((EW, Q), jnp.float32),
        pltpu.VMEM((EW, Q), jnp.float32),
        pltpu.VMEM((ZR, Q), jnp.float32),
        pltpu.VMEM_SHARED((NP_, Q), jnp.float32),
        pltpu.SemaphoreType.DMA,
        pltpu.SemaphoreType.DMA,
        pltpu.SemaphoreType.DMA,
        pltpu.SemaphoreType.DMA,
        pltpu.SemaphoreType.DMA,
        pltpu.SemaphoreType.DMA,
    ],
    compiler_params=pltpu.CompilerParams(use_tc_tiling_on_sc=False),
)
def _prop(h4, srclo, srchi, dst2d, pout,
          src0, src1, dst0, dst1, rowsA, rowsB, zbuf, acc,
          st0, st1, gAs, gBs, sAs, sBs):
    cc = lax.axis_index("c")
    ss = lax.axis_index("s")
    sbufs = (src0, src1)
    dbufs = (dst0, dst1)
    stsems = (st0, st1)

    def zrow(i, _):
        zbuf[i, :] = _zeros16()
        return 0
    lax.fori_loop(0, ZR, zrow, 0)

    def zacc(q, _):
        pltpu.sync_copy(zbuf, acc.at[pl.ds(ss * RPN + q * ZR, ZR)])
        return 0
    lax.fori_loop(0, RPN // ZR, zacc, 0)
    plsc.subcore_barrier()

    def stage(k, p):
        base = ss * WPT + k * 2

        @pl.when(cc == 0)
        def _():
            pltpu.async_copy(srclo.at[pl.ds(base, 2)], sbufs[p], stsems[p])

        @pl.when(cc == 1)
        def _():
            pltpu.async_copy(srchi.at[pl.ds(base, 2)], sbufs[p], stsems[p])

        pltpu.async_copy(dst2d.at[pl.ds(base, 2)], dbufs[p], stsems[p])

    def stage_wait(p):
        for _ in range(2):
            pltpu.make_async_copy(dst2d.at[pl.ds(0, 2)], dbufs[p],
                                  stsems[p]).wait()

    stage(0, 0)

    def body(g, _):
        for p in range(2):
            k = g * 2 + p
            sb, db = sbufs[p], dbufs[p]
            stage_wait(p)
            iA = sb.at[0]
            iB = sb.at[1]
            oA = db.at[0]
            oB = db.at[1]
            pltpu.async_copy(h4.at[iA], rowsA, gAs)
            pltpu.async_copy(h4.at[iB], rowsB, gBs)

            @pl.when(k + 1 < NPCH)
            def _():
                stage(k + 1, 1 - p)

            pltpu.make_async_copy(h4.at[iA], rowsA, gAs).wait()
            pltpu.async_copy(rowsA, acc.at[oA], sAs, add=True)
            pltpu.make_async_copy(h4.at[iB], rowsB, gBs).wait()
            pltpu.async_copy(rowsB, acc.at[oB], sBs, add=True)
            pltpu.make_async_copy(rowsA, acc.at[oA], sAs).wait()
            pltpu.make_async_copy(rowsB, acc.at[oB], sBs).wait()
        return 0
    lax.fori_loop(0, NPCH // 2, body, 0)
    plsc.subcore_barrier()

    pltpu.sync_copy(acc.at[pl.ds(ss * RPN, RPN)],
                    pout.at[pl.ds(cc * NP_ + ss * RPN, RPN)])


def _embed_body(nt, xc, xg, xp, xr, xv, d0, d1, W1, b1, W2, b2, W0,
                h0, h1, h2, h3, dinv):
    oh = (nt[...] == lax.broadcasted_iota(jnp.int32, (RB, T), 1))
    e1 = jnp.dot(oh.astype(jnp.float32), W1[...],
                 preferred_element_type=jnp.float32) + b1[...]
    w2 = W2[...]
    e2 = (xc[...] * w2[0:1, :] + xg[...] * w2[1:2, :] + xp[...] * w2[2:3, :]
          + xr[...] * w2[3:4, :] + xv[...] * w2[4:5, :]) + b2[...]
    z = jnp.concatenate([e1, e2], axis=1)
    deg = 1.0 + d0[...][:, 0:1] + d1[...][:, 0:1]
    di = lax.rsqrt(deg)
    h = jnp.dot(z, W0[...], preferred_element_type=jnp.float32) * di
    h0[...] = h[:, 0 * Q:1 * Q]
    h1[...] = h[:, 1 * Q:2 * Q]
    h2[...] = h[:, 2 * Q:3 * Q]
    h3[...] = h[:, 3 * Q:4 * Q]
    dinv[...] = di


def _embed_call(nt2, cols, d0, d1, W1, b1r, W2, b2r, W0):
    col = pl.BlockSpec((RB, 1), lambda i: (i, 0))
    dq = pl.BlockSpec((RB, Q), lambda i: (i, 0))
    full = lambda s: pl.BlockSpec(s, lambda i: (0, 0))
    qs = jax.ShapeDtypeStruct((NP_, Q), jnp.float32)
    return pl.pallas_call(
        _embed_body,
        grid=(GRID,),
        in_specs=[col, col, col, col, col, col, dq, dq,
                  full((T, H)), full((1, H)), full((5, H)), full((1, H)),
                  full((2 * H, H))],
        out_specs=[dq, dq, dq, dq, col],
        out_shape=[qs, qs, qs, qs,
                   jax.ShapeDtypeStruct((NP_, 1), jnp.float32)],
    )(nt2, *cols, d0, d1, W1, b1r, W2, b2r, W0)


def _mid_body(p0, p1, p2, p3, h0, h1, h2, h3, dinv, bb, Wn,
              o0, o1, o2, o3):
    di = dinv[...]
    z = jnp.concatenate(
        [p0[...] + h0[...], p1[...] + h1[...],
         p2[...] + h2[...], p3[...] + h3[...]], axis=1)
    z = di * z + bb[...]
    z = jnp.maximum(z, 0.0)
    h = jnp.dot(z, Wn[...], preferred_element_type=jnp.float32) * di
    o0[...] = h[:, 0 * Q:1 * Q]
    o1[...] = h[:, 1 * Q:2 * Q]
    o2[...] = h[:, 2 * Q:3 * Q]
    o3[...] = h[:, 3 * Q:4 * Q]


def _mid_call(poutA, poutB, hq, dinv2, bb, Wn):
    dq = pl.BlockSpec((RB, Q), lambda i: (i, 0))
    pLO = pl.BlockSpec((RB, Q), lambda i: (i, 0))
    pHI = pl.BlockSpec((RB, Q), lambda i: (GRID + i, 0))
    col = pl.BlockSpec((RB, 1), lambda i: (i, 0))
    full = lambda s: pl.BlockSpec(s, lambda i: (0, 0))
    qs = jax.ShapeDtypeStruct((NP_, Q), jnp.float32)
    return pl.pallas_call(
        _mid_body,
        grid=(GRID,),
        in_specs=[pLO, pHI, pLO, pHI, dq, dq, dq, dq, col,
                  full((1, H)), full((H, H))],
        out_specs=[dq, dq, dq, dq],
        out_shape=[qs, qs, qs, qs],
    )(poutA, poutA, poutB, poutB, *hq, dinv2, bb, Wn)


def _fin_body(p0, p1, p2, p3, h0, h1, h2, h3, dinv, bb, bt, Wo, bo,
              accum, pred):
    i = pl.program_id(0)
    di = dinv[...]
    z = jnp.concatenate(
        [p0[...] + h0[...], p1[...] + h1[...],
         p2[...] + h2[...], p3[...] + h3[...]], axis=1)
    z = di * z + bb[...]
    m = (bt[...] == lax.broadcasted_iota(jnp.int32, (RB, NG), 1))
    zaug = jnp.concatenate(
        [z, jnp.ones((RB, 1), jnp.float32), jnp.zeros((RB, 63), jnp.float32)],
        axis=1)
    contrib = lax.dot_general(m.astype(jnp.float32), zaug,
                              (((0,), (0,)), ((), ())),
                              preferred_element_type=jnp.float32)

    @pl.when(i == 0)
    def _():
        accum[...] = jnp.zeros((NG, 128), jnp.float32)

    accum[...] += contrib

    @pl.when(i == GRID - 1)
    def _():
        a = accum[...]
        pooled = a[:, :H] / jnp.maximum(a[:, H:H + 1], 1.0)
        pred[...] = jnp.dot(pooled, Wo[...],
                            preferred_element_type=jnp.float32) + bo[...]


def _fin_call(poutA, poutB, hq, dinv2, bb, bt2, Wop, bop):
    dq = pl.BlockSpec((RB, Q), lambda i: (i, 0))
    pLO = pl.BlockSpec((RB, Q), lambda i: (i, 0))
    pHI = pl.BlockSpec((RB, Q), lambda i: (GRID + i, 0))
    col = pl.BlockSpec((RB, 1), lambda i: (i, 0))
    full = lambda s: pl.BlockSpec(s, lambda i: (0, 0))
    acc_spec = pl.BlockSpec((NG, 128), lambda i: (0, 0))
    _, pred = pl.pallas_call(
        _fin_body,
        grid=(GRID,),
        in_specs=[pLO, pHI, pLO, pHI, dq, dq, dq, dq, col,
                  full((1, H)), col, full((H, 128)), full((1, 128))],
        out_specs=[acc_spec, acc_spec],
        out_shape=[jax.ShapeDtypeStruct((NG, 128), jnp.float32),
                   jax.ShapeDtypeStruct((NG, 128), jnp.float32)],
    )(poutA, poutA, poutB, poutB, *hq, dinv2, bb, bt2, Wop, bop)
    return pred


def kernel(node_type, c, gm, pos, r, vid, edge_index, batch,
           W1, b1, W2, b2, gcn_params, Wout, bout):
    f32, i32 = jnp.float32, jnp.int32

    src = edge_index[0].astype(i32)
    dst = edge_index[1].astype(i32)
    src2d = jnp.concatenate([src, jnp.zeros((EP - E,), i32)]).reshape(EPW, EW)
    dst2d = jnp.concatenate([dst, jnp.full((EP - E,), N, i32)]).reshape(EPW, EW)
    srcq = [src2d + k * NP_ for k in range(4)]
    ones2d = jnp.ones((EW, Q), f32)

    deg0, deg1 = _deg(dst2d, ones2d)

    padc = lambda a: jnp.pad(a.astype(f32), (0, NP_ - N)).reshape(NP_, 1)
    nt2 = jnp.pad(node_type.astype(i32), (0, NP_ - N)).reshape(NP_, 1)
    cols = [padc(c), padc(gm), padc(pos), padc(r), padc(vid)]
    bt2 = jnp.pad(batch.astype(i32), (0, NP_ - N),
                  constant_values=1 << 20).reshape(NP_, 1)

    W0 = gcn_params[0][0]
    *hq, dinv2 = _embed_call(nt2, cols, deg0, deg1,
                             W1, b1.reshape(1, H), W2, b2.reshape(1, H), W0)

    for l in range(NL):
        h4 = jnp.concatenate(hq, axis=0)
        poutA = _prop(h4, srcq[0], srcq[1], dst2d)
        poutB = _prop(h4, srcq[2], srcq[3], dst2d)
        bb = gcn_params[l][1].reshape(1, H)
        if l < NL - 1:
            hq = _mid_call(poutA, poutB, hq, dinv2, bb, gcn_params[l + 1][0])
        else:
            Wop = jnp.pad(Wout.astype(f32), ((0, 0), (0, 128 - 4)))
            bop = jnp.pad(bout.astype(f32), (0, 128 - 4)).reshape(1, 128)
            pred = _fin_call(poutA, poutB, hq, dinv2, bb, bt2, Wop, bop)

    return pred[:, :4]


# speedup vs baseline: 1.0391x; 1.0391x over previous
import functools

import jax
import jax.numpy as jnp
from jax import lax
from jax.experimental import pallas as pl
from jax.experimental.pallas import tpu as pltpu
from jax.experimental.pallas import tpu_sc as plsc

N = 50000
E = 800000
H = 64
T = 16
NG = 128
NL = 5
Q = 16

RPT_ALL = 200
EPR = 32 * RPT_ALL
EP = EPR * 128
RPT_SC = EPR // 16
NP_ = 50176
RPN = NP_ // 16
ZR = 112
RB = 512
GRID = NP_ // RB
EW = 1280
EPW = EP // EW
WPT = EPW // 16
WPT_ALL = EPW // 32

_mesh = plsc.VectorSubcoreMesh(core_axis_name="c", subcore_axis_name="s")


def _zeros16():
    return jnp.zeros((16,), jnp.float32)


@functools.partial(
    pl.kernel,
    mesh=_mesh,
    out_type=[jax.ShapeDtypeStruct((NP_, Q), jnp.float32),
              jax.ShapeDtypeStruct((NP_, Q), jnp.float32)],
    scratch_types=[
        pltpu.VMEM((WPT_ALL, EW), jnp.int32),
        pltpu.VMEM((EW, Q), jnp.float32),
        pltpu.VMEM((ZR, Q), jnp.float32),
        pltpu.VMEM_SHARED((NP_, Q), jnp.float32),
        pltpu.SemaphoreType.DMA,
    ],
    compiler_params=pltpu.CompilerParams(use_tc_tiling_on_sc=False),
)
def _deg(dst2d, ones2d, out0, out1, dstbuf, onesbuf, zbuf, acc, sem):
    cc = lax.axis_index("c")
    ss = lax.axis_index("s")
    wid = ss * 2 + cc

    pltpu.sync_copy(dst2d.at[pl.ds(wid * WPT_ALL, WPT_ALL)], dstbuf)
    pltpu.sync_copy(ones2d, onesbuf)

    def zrow(i, _):
        zbuf[i, :] = _zeros16()
        return 0
    lax.fori_loop(0, ZR, zrow, 0)

    def zacc(q, _):
        pltpu.sync_copy(zbuf, acc.at[pl.ds(ss * RPN + q * ZR, ZR)])
        return 0
    lax.fori_loop(0, RPN // ZR, zacc, 0)
    plsc.subcore_barrier()

    def body(k, _):
        pltpu.async_copy(onesbuf, acc.at[dstbuf.at[k]], sem, add=True)
        @pl.when(k >= 2)
        def _():
            pltpu.make_async_copy(onesbuf, acc.at[dstbuf.at[0]],
                                  sem).wait()
        return 0
    lax.fori_loop(0, WPT_ALL, body, 0)
    for b in range(2):
        pltpu.make_async_copy(onesbuf, acc.at[dstbuf.at[0]], sem).wait()
    plsc.subcore_barrier()

    @pl.when(cc == 0)
    def _():
        pltpu.sync_copy(acc.at[pl.ds(ss * RPN, RPN)],
                        out0.at[pl.ds(ss * RPN, RPN)])

    @pl.when(cc == 1)
    def _():
        pltpu.sync_copy(acc.at[pl.ds(ss * RPN, RPN)],
                        out1.at[pl.ds(ss * RPN, RPN)])


@functools.partial(
    pl.kernel,
    mesh=_mesh,
    out_type=jax.ShapeDtypeStruct((2 * NP_, Q), jnp.float32),
    scratch_types=[
        pltpu.VMEM((1, EW), jnp.int32),
        pltpu.VMEM((1, EW), jnp.int32),
        pltpu.VMEM((1, EW), jnp.int32),
        pltpu.VMEM((1, EW), jnp.int32),
        pltpu.VMEM((1, EW), jnp.int32),
        pltpu.VMEM((1, EW), jnp.int32),
        pltpu.VMEM((1, EW), jnp.int32),
        pltpu.VMEM((1, EW), jnp.int32),
        pltpu.VMEM((EW, Q), jnp.float32),
        pltpu.VMEM((EW, Q), jnp.float32),
        pltpu.VMEM((ZR, Q), jnp.float32),
        pltpu.VMEM_SHARED((NP_, Q), jnp.float32),
        pltpu.SemaphoreType.DMA,
        pltpu.SemaphoreType.DMA,
        pltpu.SemaphoreType.DMA,
        pltpu.SemaphoreType.DMA,
        pltpu.SemaphoreType.DMA,
        pltpu.SemaphoreType.DMA,
        pltpu.SemaphoreType.DMA,
        pltpu.SemaphoreType.DMA,
    ],
    compiler_params=pltpu.CompilerParams(use_tc_tiling_on_sc=False),
)
def _prop(h4, srclo, srchi, dst2d, pout,
          sb0, sb1, sb2, sb3, db0, db1, db2, db3, rw0, rw1, zbuf, acc,
          st0, st1, st2, st3, gs0, gs1, ss0, ss1):
    cc = lax.axis_index("c")
    ss = lax.axis_index("s")
    sbufs = (sb0, sb1, sb2, sb3)
    dbufs = (db0, db1, db2, db3)
    stsems = (st0, st1, st2, st3)
    rows = (rw0, rw1)
    gsems = (gs0, gs1)
    ssems = (ss0, ss1)

    def zrow(i, _):
        zbuf[i, :] = _zeros16()
        return 0
    lax.fori_loop(0, ZR, zrow, 0)

    def zacc(q, _):
        pltpu.sync_copy(zbuf, acc.at[pl.ds(ss * RPN + q * ZR, ZR)])
        return 0
    lax.fori_loop(0, RPN // ZR, zacc, 0)
    plsc.subcore_barrier()

    def stage(u, pi):
        base = ss * WPT + u

        @pl.when(cc == 0)
        def _():
            pltpu.async_copy(srclo.at[pl.ds(base, 1)], sbufs[pi], stsems[pi])

        @pl.when(cc == 1)
        def _():
            pltpu.async_copy(srchi.at[pl.ds(base, 1)], sbufs[pi], stsems[pi])

        pltpu.async_copy(dst2d.at[pl.ds(base, 1)], dbufs[pi], stsems[pi])

    def unit(u, pr, pi, first, last):
        if not first:
            @pl.when(u >= 2)
            def _():
                pltpu.make_async_copy(
                    rows[pr], acc.at[dbufs[pi].at[0]], ssems[pr]).wait()
        if not last:
            @pl.when(u + 2 < WPT)
            def _():
                stage(u + 2, (pi + 2) % 4)
        for _ in range(2):
            pltpu.make_async_copy(dst2d.at[pl.ds(0, 1)], dbufs[pi],
                                  stsems[pi]).wait()
        gi = h4.at[sbufs[pi].at[0]]
        pltpu.async_copy(gi, rows[pr], gsems[pr])
        pltpu.make_async_copy(gi, rows[pr], gsems[pr]).wait()
        pltpu.async_copy(rows[pr], acc.at[dbufs[pi].at[0]], ssems[pr],
                         add=True)

    stage(0, 0)
    stage(1, 1)

    def body(g, _):
        for uu in range(4):
            unit(g * 4 + uu, uu % 2, uu % 4, False, False)
        return 0
    lax.fori_loop(0, WPT // 4, body, 0)
    for p in range(2):
        pltpu.make_async_copy(rows[p], acc.at[dbufs[p].at[0]],
                              ssems[p]).wait()
    plsc.subcore_barrier()

    pltpu.sync_copy(acc.at[pl.ds(ss * RPN, RPN)],
                    pout.at[pl.ds(cc * NP_ + ss * RPN, RPN)])


def _embed_body(nt, xc, xg, xp, xr, xv, d0, d1, W1, b1, W2, b2, W0,
                h0, h1, h2, h3, dinv):
    oh = (nt[...] == lax.broadcasted_iota(jnp.int32, (RB, T), 1))
    e1 = jnp.dot(oh.astype(jnp.float32), W1[...],
                 preferred_element_type=jnp.float32) + b1[...]
    w2 = W2[...]
    e2 = (xc[...] * w2[0:1, :] + xg[...] * w2[1:2, :] + xp[...] * w2[2:3, :]
          + xr[...] * w2[3:4, :] + xv[...] * w2[4:5, :]) + b2[...]
    z = jnp.concatenate([e1, e2], axis=1)
    deg = 1.0 + d0[...][:, 0:1] + d1[...][:, 0:1]
    di = lax.rsqrt(deg)
    h = jnp.dot(z, W0[...], preferred_element_type=jnp.float32) * di
    h0[...] = h[:, 0 * Q:1 * Q]
    h1[...] = h[:, 1 * Q:2 * Q]
    h2[...] = h[:, 2 * Q:3 * Q]
    h3[...] = h[:, 3 * Q:4 * Q]
    dinv[...] = di


def _embed_call(nt2, cols, d0, d1, W1, b1r, W2, b2r, W0):
    col = pl.BlockSpec((RB, 1), lambda i: (i, 0))
    dq = pl.BlockSpec((RB, Q), lambda i: (i, 0))
    full = lambda s: pl.BlockSpec(s, lambda i: (0, 0))
    qs = jax.ShapeDtypeStruct((NP_, Q), jnp.float32)
    return pl.pallas_call(
        _embed_body,
        grid=(GRID,),
        in_specs=[col, col, col, col, col, col, dq, dq,
                  full((T, H)), full((1, H)), full((5, H)), full((1, H)),
                  full((2 * H, H))],
        out_specs=[dq, dq, dq, dq, col],
        out_shape=[qs, qs, qs, qs,
                   jax.ShapeDtypeStruct((NP_, 1), jnp.float32)],
    )(nt2, *cols, d0, d1, W1, b1r, W2, b2r, W0)


def _mid_body(p0, p1, p2, p3, h0, h1, h2, h3, dinv, bb, Wn,
              o0, o1, o2, o3):
    di = dinv[...]
    z = jnp.concatenate(
        [p0[...] + h0[...], p1[...] + h1[...],
         p2[...] + h2[...], p3[...] + h3[...]], axis=1)
    z = di * z + bb[...]
    z = jnp.maximum(z, 0.0)
    h = jnp.dot(z, Wn[...], preferred_element_type=jnp.float32) * di
    o0[...] = h[:, 0 * Q:1 * Q]
    o1[...] = h[:, 1 * Q:2 * Q]
    o2[...] = h[:, 2 * Q:3 * Q]
    o3[...] = h[:, 3 * Q:4 * Q]


def _mid_call(poutA, poutB, hq, dinv2, bb, Wn):
    dq = pl.BlockSpec((RB, Q), lambda i: (i, 0))
    pLO = pl.BlockSpec((RB, Q), lambda i: (i, 0))
    pHI = pl.BlockSpec((RB, Q), lambda i: (GRID + i, 0))
    col = pl.BlockSpec((RB, 1), lambda i: (i, 0))
    full = lambda s: pl.BlockSpec(s, lambda i: (0, 0))
    qs = jax.ShapeDtypeStruct((NP_, Q), jnp.float32)
    return pl.pallas_call(
        _mid_body,
        grid=(GRID,),
        in_specs=[pLO, pHI, pLO, pHI, dq, dq, dq, dq, col,
                  full((1, H)), full((H, H))],
        out_specs=[dq, dq, dq, dq],
        out_shape=[qs, qs, qs, qs],
    )(poutA, poutA, poutB, poutB, *hq, dinv2, bb, Wn)


def _fin_body(p0, p1, p2, p3, h0, h1, h2, h3, dinv, bb, bt, Wo, bo,
              accum, pred):
    i = pl.program_id(0)
    di = dinv[...]
    z = jnp.concatenate(
        [p0[...] + h0[...], p1[...] + h1[...],
         p2[...] + h2[...], p3[...] + h3[...]], axis=1)
    z = di * z + bb[...]
    m = (bt[...] == lax.broadcasted_iota(jnp.int32, (RB, NG), 1))
    zaug = jnp.concatenate(
        [z, jnp.ones((RB, 1), jnp.float32), jnp.zeros((RB, 63), jnp.float32)],
        axis=1)
    contrib = lax.dot_general(m.astype(jnp.float32), zaug,
                              (((0,), (0,)), ((), ())),
                              preferred_element_type=jnp.float32)

    @pl.when(i == 0)
    def _():
        accum[...] = jnp.zeros((NG, 128), jnp.float32)

    accum[...] += contrib

    @pl.when(i == GRID - 1)
    def _():
        a = accum[...]
        pooled = a[:, :H] / jnp.maximum(a[:, H:H + 1], 1.0)
        pred[...] = jnp.dot(pooled, Wo[...],
                            preferred_element_type=jnp.float32) + bo[...]


def _fin_call(poutA, poutB, hq, dinv2, bb, bt2, Wop, bop):
    dq = pl.BlockSpec((RB, Q), lambda i: (i, 0))
    pLO = pl.BlockSpec((RB, Q), lambda i: (i, 0))
    pHI = pl.BlockSpec((RB, Q), lambda i: (GRID + i, 0))
    col = pl.BlockSpec((RB, 1), lambda i: (i, 0))
    full = lambda s: pl.BlockSpec(s, lambda i: (0, 0))
    acc_spec = pl.BlockSpec((NG, 128), lambda i: (0, 0))
    _, pred = pl.pallas_call(
        _fin_body,
        grid=(GRID,),
        in_specs=[pLO, pHI, pLO, pHI, dq, dq, dq, dq, col,
                  full((1, H)), col, full((H, 128)), full((1, 128))],
        out_specs=[acc_spec, acc_spec],
        out_shape=[jax.ShapeDtypeStruct((NG, 128), jnp.float32),
                   jax.ShapeDtypeStruct((NG, 128), jnp.float32)],
    )(poutA, poutA, poutB, poutB, *hq, dinv2, bb, bt2, Wop, bop)
    return pred


def kernel(node_type, c, gm, pos, r, vid, edge_index, batch,
           W1, b1, W2, b2, gcn_params, Wout, bout):
    f32, i32 = jnp.float32, jnp.int32

    src = edge_index[0].astype(i32)
    dst = edge_index[1].astype(i32)
    src2d = jnp.concatenate([src, jnp.zeros((EP - E,), i32)]).reshape(EPW, EW)
    dst2d = jnp.concatenate([dst, jnp.full((EP - E,), N, i32)]).reshape(EPW, EW)
    srcq = [src2d + k * NP_ for k in range(4)]
    ones2d = jnp.ones((EW, Q), f32)

    deg0, deg1 = _deg(dst2d, ones2d)

    padc = lambda a: jnp.pad(a.astype(f32), (0, NP_ - N)).reshape(NP_, 1)
    nt2 = jnp.pad(node_type.astype(i32), (0, NP_ - N)).reshape(NP_, 1)
    cols = [padc(c), padc(gm), padc(pos), padc(r), padc(vid)]
    bt2 = jnp.pad(batch.astype(i32), (0, NP_ - N),
                  constant_values=1 << 20).reshape(NP_, 1)

    W0 = gcn_params[0][0]
    *hq, dinv2 = _embed_call(nt2, cols, deg0, deg1,
                             W1, b1.reshape(1, H), W2, b2.reshape(1, H), W0)

    for l in range(NL):
        h4 = jnp.concatenate(hq, axis=0)
        poutA = _prop(h4, srcq[0], srcq[1], dst2d)
        poutB = _prop(h4, srcq[2], srcq[3], dst2d)
        bb = gcn_params[l][1].reshape(1, H)
        if l < NL - 1:
            hq = _mid_call(poutA, poutB, hq, dinv2, bb, gcn_params[l + 1][0])
        else:
            Wop = jnp.pad(Wout.astype(f32), ((0, 0), (0, 128 - 4)))
            bop = jnp.pad(bout.astype(f32), (0, 128 - 4)).reshape(1, 128)
            pred = _fin_call(poutA, poutB, hq, dinv2, bb, bt2, Wop, bop)

    return pred[:, :4]


# speedup vs baseline: 1.1280x; 1.0856x over previous
import functools

import jax
import jax.numpy as jnp
from jax import lax
from jax.experimental import pallas as pl
from jax.experimental.pallas import tpu as pltpu
from jax.experimental.pallas import tpu_sc as plsc

N = 50000
E = 800000
H = 64
T = 16
NG = 128
NL = 5
Q = 16

RPT_ALL = 200
EPR = 32 * RPT_ALL
EP = EPR * 128
RPT_SC = EPR // 16
NP_ = 50176
RPN = NP_ // 16
ZR = 112
RB = 512
GRID = NP_ // RB
EW = 1280
EPW = EP // EW
WPT = EPW // 16
WPT_ALL = EPW // 32

_mesh = plsc.VectorSubcoreMesh(core_axis_name="c", subcore_axis_name="s")


def _zeros16():
    return jnp.zeros((16,), jnp.float32)


@functools.partial(
    pl.kernel,
    mesh=_mesh,
    out_type=[jax.ShapeDtypeStruct((NP_, Q), jnp.float32),
              jax.ShapeDtypeStruct((NP_, Q), jnp.float32)],
    scratch_types=[
        pltpu.VMEM((WPT_ALL, EW), jnp.int32),
        pltpu.VMEM((EW, Q), jnp.float32),
        pltpu.VMEM((ZR, Q), jnp.float32),
        pltpu.VMEM_SHARED((NP_, Q), jnp.float32),
        pltpu.SemaphoreType.DMA,
    ],
    compiler_params=pltpu.CompilerParams(use_tc_tiling_on_sc=False),
)
def _deg(dst2d, ones2d, out0, out1, dstbuf, onesbuf, zbuf, acc, sem):
    cc = lax.axis_index("c")
    ss = lax.axis_index("s")
    wid = ss * 2 + cc

    pltpu.sync_copy(dst2d.at[pl.ds(wid * WPT_ALL, WPT_ALL)], dstbuf)
    pltpu.sync_copy(ones2d, onesbuf)

    def zrow(i, _):
        zbuf[i, :] = _zeros16()
        return 0
    lax.fori_loop(0, ZR, zrow, 0)

    def zacc(q, _):
        pltpu.sync_copy(zbuf, acc.at[pl.ds(ss * RPN + q * ZR, ZR)])
        return 0
    lax.fori_loop(0, RPN // ZR, zacc, 0)
    plsc.subcore_barrier()

    def body(k, _):
        pltpu.async_copy(onesbuf, acc.at[dstbuf.at[k]], sem, add=True)
        @pl.when(k >= 2)
        def _():
            pltpu.make_async_copy(onesbuf, acc.at[dstbuf.at[0]],
                                  sem).wait()
        return 0
    lax.fori_loop(0, WPT_ALL, body, 0)
    for b in range(2):
        pltpu.make_async_copy(onesbuf, acc.at[dstbuf.at[0]], sem).wait()
    plsc.subcore_barrier()

    @pl.when(cc == 0)
    def _():
        pltpu.sync_copy(acc.at[pl.ds(ss * RPN, RPN)],
                        out0.at[pl.ds(ss * RPN, RPN)])

    @pl.when(cc == 1)
    def _():
        pltpu.sync_copy(acc.at[pl.ds(ss * RPN, RPN)],
                        out1.at[pl.ds(ss * RPN, RPN)])


@functools.partial(
    pl.kernel,
    mesh=_mesh,
    out_type=jax.ShapeDtypeStruct((4 * NP_, Q), jnp.float32),
    scratch_types=[
        pltpu.VMEM((1, EW), jnp.int32),
        pltpu.VMEM((1, EW), jnp.int32),
        pltpu.VMEM((1, EW), jnp.int32),
        pltpu.VMEM((1, EW), jnp.int32),
        pltpu.VMEM((1, EW), jnp.int32),
        pltpu.VMEM((1, EW), jnp.int32),
        pltpu.VMEM((1, EW), jnp.int32),
        pltpu.VMEM((1, EW), jnp.int32),
        pltpu.VMEM((EW, Q), jnp.float32),
        pltpu.VMEM((EW, Q), jnp.float32),
        pltpu.VMEM((ZR, Q), jnp.float32),
        pltpu.VMEM_SHARED((NP_, Q), jnp.float32),
        pltpu.SemaphoreType.DMA,
        pltpu.SemaphoreType.DMA,
        pltpu.SemaphoreType.DMA,
        pltpu.SemaphoreType.DMA,
        pltpu.SemaphoreType.DMA,
        pltpu.SemaphoreType.DMA,
        pltpu.SemaphoreType.DMA,
        pltpu.SemaphoreType.DMA,
    ],
    compiler_params=pltpu.CompilerParams(use_tc_tiling_on_sc=False),
)
def _prop(h0, h1, h2, h3, src2d, dst2d, pout,
          sb0, sb1, sb2, sb3, db0, db1, db2, db3, rw0, rw1, zbuf, acc,
          st0, st1, st2, st3, gs0, gs1, ss0, ss1):
    cc = lax.axis_index("c")
    ss = lax.axis_index("s")
    sbufs = (sb0, sb1, sb2, sb3)
    dbufs = (db0, db1, db2, db3)
    stsems = (st0, st1, st2, st3)
    rows = (rw0, rw1)
    gsems = (gs0, gs1)
    ssems = (ss0, ss1)

    def zrow(i, _):
        zbuf[i, :] = _zeros16()
        return 0
    lax.fori_loop(0, ZR, zrow, 0)

    def stage(u, pi):
        base = ss * WPT + u
        pltpu.async_copy(src2d.at[pl.ds(base, 1)], sbufs[pi], stsems[pi])
        pltpu.async_copy(dst2d.at[pl.ds(base, 1)], dbufs[pi], stsems[pi])

    for pas in range(2):
        tabs = (h0, h1) if pas == 0 else (h2, h3)

        def zacc(q, _):
            pltpu.sync_copy(zbuf, acc.at[pl.ds(ss * RPN + q * ZR, ZR)])
            return 0
        lax.fori_loop(0, RPN // ZR, zacc, 0)
        plsc.subcore_barrier()

        def unit(u, pr, pi):
            @pl.when(u >= 2)
            def _():
                pltpu.make_async_copy(
                    rows[pr], acc.at[dbufs[pi].at[0]], ssems[pr]).wait()

            @pl.when(u + 2 < WPT)
            def _():
                stage(u + 2, (pi + 2) % 4)

            for _ in range(2):
                pltpu.make_async_copy(dst2d.at[pl.ds(0, 1)], dbufs[pi],
                                      stsems[pi]).wait()

            @pl.when(cc == 0)
            def _():
                pltpu.async_copy(tabs[0].at[sbufs[pi].at[0]], rows[pr],
                                 gsems[pr])

            @pl.when(cc == 1)
            def _():
                pltpu.async_copy(tabs[1].at[sbufs[pi].at[0]], rows[pr],
                                 gsems[pr])

            pltpu.make_async_copy(tabs[0].at[sbufs[pi].at[0]], rows[pr],
                                  gsems[pr]).wait()
            pltpu.async_copy(rows[pr], acc.at[dbufs[pi].at[0]], ssems[pr],
                             add=True)

        stage(0, 0)
        stage(1, 1)

        def body(g, _):
            for uu in range(4):
                unit(g * 4 + uu, uu % 2, uu % 4)
            return 0
        lax.fori_loop(0, WPT // 4, body, 0)
        for p in range(2):
            pltpu.make_async_copy(rows[p], acc.at[dbufs[p].at[0]],
                                  ssems[p]).wait()
        plsc.subcore_barrier()

        pltpu.sync_copy(acc.at[pl.ds(ss * RPN, RPN)],
                        pout.at[pl.ds((pas * 2 + cc) * NP_ + ss * RPN, RPN)])
        plsc.subcore_barrier()


def _embed_body(nt, xc, xg, xp, xr, xv, d0, d1, W1, b1, W2, b2, W0,
                h0, h1, h2, h3, dinv):
    oh = (nt[...] == lax.broadcasted_iota(jnp.int32, (RB, T), 1))
    e1 = jnp.dot(oh.astype(jnp.float32), W1[...],
                 preferred_element_type=jnp.float32) + b1[...]
    w2 = W2[...]
    e2 = (xc[...] * w2[0:1, :] + xg[...] * w2[1:2, :] + xp[...] * w2[2:3, :]
          + xr[...] * w2[3:4, :] + xv[...] * w2[4:5, :]) + b2[...]
    z = jnp.concatenate([e1, e2], axis=1)
    deg = 1.0 + d0[...][:, 0:1] + d1[...][:, 0:1]
    di = lax.rsqrt(deg)
    h = jnp.dot(z, W0[...], preferred_element_type=jnp.float32) * di
    h0[...] = h[:, 0 * Q:1 * Q]
    h1[...] = h[:, 1 * Q:2 * Q]
    h2[...] = h[:, 2 * Q:3 * Q]
    h3[...] = h[:, 3 * Q:4 * Q]
    dinv[...] = di


def _embed_call(nt2, cols, d0, d1, W1, b1r, W2, b2r, W0):
    col = pl.BlockSpec((RB, 1), lambda i: (i, 0))
    dq = pl.BlockSpec((RB, Q), lambda i: (i, 0))
    full = lambda s: pl.BlockSpec(s, lambda i: (0, 0))
    qs = jax.ShapeDtypeStruct((NP_, Q), jnp.float32)
    return pl.pallas_call(
        _embed_body,
        grid=(GRID,),
        in_specs=[col, col, col, col, col, col, dq, dq,
                  full((T, H)), full((1, H)), full((5, H)), full((1, H)),
                  full((2 * H, H))],
        out_specs=[dq, dq, dq, dq, col],
        out_shape=[qs, qs, qs, qs,
                   jax.ShapeDtypeStruct((NP_, 1), jnp.float32)],
    )(nt2, *cols, d0, d1, W1, b1r, W2, b2r, W0)


def _mid_body(p0, p1, p2, p3, h0, h1, h2, h3, dinv, bb, Wn,
              o0, o1, o2, o3):
    di = dinv[...]
    z = jnp.concatenate(
        [p0[...] + h0[...], p1[...] + h1[...],
         p2[...] + h2[...], p3[...] + h3[...]], axis=1)
    z = di * z + bb[...]
    z = jnp.maximum(z, 0.0)
    h = jnp.dot(z, Wn[...], preferred_element_type=jnp.float32) * di
    o0[...] = h[:, 0 * Q:1 * Q]
    o1[...] = h[:, 1 * Q:2 * Q]
    o2[...] = h[:, 2 * Q:3 * Q]
    o3[...] = h[:, 3 * Q:4 * Q]


def _mid_call(pout, hq, dinv2, bb, Wn):
    dq = pl.BlockSpec((RB, Q), lambda i: (i, 0))
    pq = [pl.BlockSpec((RB, Q), lambda i, k=k: (k * GRID + i, 0))
          for k in range(4)]
    col = pl.BlockSpec((RB, 1), lambda i: (i, 0))
    full = lambda s: pl.BlockSpec(s, lambda i: (0, 0))
    qs = jax.ShapeDtypeStruct((NP_, Q), jnp.float32)
    return pl.pallas_call(
        _mid_body,
        grid=(GRID,),
        in_specs=[*pq, dq, dq, dq, dq, col,
                  full((1, H)), full((H, H))],
        out_specs=[dq, dq, dq, dq],
        out_shape=[qs, qs, qs, qs],
    )(pout, pout, pout, pout, *hq, dinv2, bb, Wn)


def _fin_body(p0, p1, p2, p3, h0, h1, h2, h3, dinv, bb, bt, Wo, bo,
              accum, pred):
    i = pl.program_id(0)
    di = dinv[...]
    z = jnp.concatenate(
        [p0[...] + h0[...], p1[...] + h1[...],
         p2[...] + h2[...], p3[...] + h3[...]], axis=1)
    z = di * z + bb[...]
    m = (bt[...] == lax.broadcasted_iota(jnp.int32, (RB, NG), 1))
    zaug = jnp.concatenate(
        [z, jnp.ones((RB, 1), jnp.float32), jnp.zeros((RB, 63), jnp.float32)],
        axis=1)
    contrib = lax.dot_general(m.astype(jnp.float32), zaug,
                              (((0,), (0,)), ((), ())),
                              preferred_element_type=jnp.float32)

    @pl.when(i == 0)
    def _():
        accum[...] = jnp.zeros((NG, 128), jnp.float32)

    accum[...] += contrib

    @pl.when(i == GRID - 1)
    def _():
        a = accum[...]
        pooled = a[:, :H] / jnp.maximum(a[:, H:H + 1], 1.0)
        pred[...] = jnp.dot(pooled, Wo[...],
                            preferred_element_type=jnp.float32) + bo[...]


def _fin_call(pout, hq, dinv2, bb, bt2, Wop, bop):
    dq = pl.BlockSpec((RB, Q), lambda i: (i, 0))
    pq = [pl.BlockSpec((RB, Q), lambda i, k=k: (k * GRID + i, 0))
          for k in range(4)]
    col = pl.BlockSpec((RB, 1), lambda i: (i, 0))
    full = lambda s: pl.BlockSpec(s, lambda i: (0, 0))
    acc_spec = pl.BlockSpec((NG, 128), lambda i: (0, 0))
    _, pred = pl.pallas_call(
        _fin_body,
        grid=(GRID,),
        in_specs=[*pq, dq, dq, dq, dq, col,
                  full((1, H)), col, full((H, 128)), full((1, 128))],
        out_specs=[acc_spec, acc_spec],
        out_shape=[jax.ShapeDtypeStruct((NG, 128), jnp.float32),
                   jax.ShapeDtypeStruct((NG, 128), jnp.float32)],
    )(pout, pout, pout, pout, *hq, dinv2, bb, bt2, Wop, bop)
    return pred


def kernel(node_type, c, gm, pos, r, vid, edge_index, batch,
           W1, b1, W2, b2, gcn_params, Wout, bout):
    f32, i32 = jnp.float32, jnp.int32

    src = edge_index[0].astype(i32)
    dst = edge_index[1].astype(i32)
    src2d = jnp.concatenate([src, jnp.zeros((EP - E,), i32)]).reshape(EPW, EW)
    dst2d = jnp.concatenate([dst, jnp.full((EP - E,), N, i32)]).reshape(EPW, EW)
    ones2d = jnp.ones((EW, Q), f32)

    deg0, deg1 = _deg(dst2d, ones2d)

    padc = lambda a: jnp.pad(a.astype(f32), (0, NP_ - N)).reshape(NP_, 1)
    nt2 = jnp.pad(node_type.astype(i32), (0, NP_ - N)).reshape(NP_, 1)
    cols = [padc(c), padc(gm), padc(pos), padc(r), padc(vid)]
    bt2 = jnp.pad(batch.astype(i32), (0, NP_ - N),
                  constant_values=1 << 20).reshape(NP_, 1)

    W0 = gcn_params[0][0]
    *hq, dinv2 = _embed_call(nt2, cols, deg0, deg1,
                             W1, b1.reshape(1, H), W2, b2.reshape(1, H), W0)

    for l in range(NL):
        pout = _prop(*hq, src2d, dst2d)
        bb = gcn_params[l][1].reshape(1, H)
        if l < NL - 1:
            hq = _mid_call(pout, hq, dinv2, bb, gcn_params[l + 1][0])
        else:
            Wop = jnp.pad(Wout.astype(f32), ((0, 0), (0, 128 - 4)))
            bop = jnp.pad(bout.astype(f32), (0, 128 - 4)).reshape(1, 128)
            pred = _fin_call(pout, hq, dinv2, bb, bt2, Wop, bop)

    return pred[:, :4]


# speedup vs baseline: 1.1820x; 1.0478x over previous
import functools

import jax
import jax.numpy as jnp
from jax import lax
from jax.experimental import pallas as pl
from jax.experimental.pallas import tpu as pltpu
from jax.experimental.pallas import tpu_sc as plsc

N = 50000
E = 800000
H = 64
T = 16
NG = 128
NL = 5
Q = 16

RPT_ALL = 200
EPR = 32 * RPT_ALL
EP = EPR * 128
RPT_SC = EPR // 16
NP_ = 50176
RPN = NP_ // 16
ZR = 112
RB = 512
GRID = NP_ // RB
EW = 640
EPW = EP // EW
WPT = EPW // 16
WPT_ALL = EPW // 32

_mesh = plsc.VectorSubcoreMesh(core_axis_name="c", subcore_axis_name="s")


def _zeros16():
    return jnp.zeros((16,), jnp.float32)


@functools.partial(
    pl.kernel,
    mesh=_mesh,
    out_type=[jax.ShapeDtypeStruct((NP_, Q), jnp.float32),
              jax.ShapeDtypeStruct((NP_, Q), jnp.float32)],
    scratch_types=[
        pltpu.VMEM((WPT_ALL, EW), jnp.int32),
        pltpu.VMEM((EW, Q), jnp.float32),
        pltpu.VMEM((ZR, Q), jnp.float32),
        pltpu.VMEM_SHARED((NP_, Q), jnp.float32),
        pltpu.SemaphoreType.DMA,
    ],
    compiler_params=pltpu.CompilerParams(use_tc_tiling_on_sc=False),
)
def _deg(dst2d, ones2d, out0, out1, dstbuf, onesbuf, zbuf, acc, sem):
    cc = lax.axis_index("c")
    ss = lax.axis_index("s")
    wid = ss * 2 + cc

    pltpu.sync_copy(dst2d.at[pl.ds(wid * WPT_ALL, WPT_ALL)], dstbuf)
    pltpu.sync_copy(ones2d, onesbuf)

    def zrow(i, _):
        zbuf[i, :] = _zeros16()
        return 0
    lax.fori_loop(0, ZR, zrow, 0)

    def zacc(q, _):
        pltpu.sync_copy(zbuf, acc.at[pl.ds(ss * RPN + q * ZR, ZR)])
        return 0
    lax.fori_loop(0, RPN // ZR, zacc, 0)
    plsc.subcore_barrier()

    def body(k, _):
        pltpu.async_copy(onesbuf, acc.at[dstbuf.at[k]], sem, add=True)
        @pl.when(k >= 2)
        def _():
            pltpu.make_async_copy(onesbuf, acc.at[dstbuf.at[0]],
                                  sem).wait()
        return 0
    lax.fori_loop(0, WPT_ALL, body, 0)
    for b in range(2):
        pltpu.make_async_copy(onesbuf, acc.at[dstbuf.at[0]], sem).wait()
    plsc.subcore_barrier()

    @pl.when(cc == 0)
    def _():
        pltpu.sync_copy(acc.at[pl.ds(ss * RPN, RPN)],
                        out0.at[pl.ds(ss * RPN, RPN)])

    @pl.when(cc == 1)
    def _():
        pltpu.sync_copy(acc.at[pl.ds(ss * RPN, RPN)],
                        out1.at[pl.ds(ss * RPN, RPN)])


@functools.partial(
    pl.kernel,
    mesh=_mesh,
    out_type=jax.ShapeDtypeStruct((4 * NP_, Q), jnp.float32),
    scratch_types=(
        [pltpu.VMEM((1, EW), jnp.int32)] * 8 +
        [pltpu.VMEM((1, EW), jnp.int32)] * 8 +
        [pltpu.VMEM((EW, Q), jnp.float32)] * 4 +
        [pltpu.VMEM((ZR, Q), jnp.float32),
         pltpu.VMEM_SHARED((NP_, Q), jnp.float32)] +
        [pltpu.SemaphoreType.DMA] * 16
    ),
    compiler_params=pltpu.CompilerParams(use_tc_tiling_on_sc=False),
)
def _prop(h0, h1, h2, h3, src2d, dst2d, pout, *sc):
    cc = lax.axis_index("c")
    ss = lax.axis_index("s")
    sbufs = sc[0:8]
    dbufs = sc[8:16]
    rows = sc[16:20]
    zbuf = sc[20]
    acc = sc[21]
    stsems = sc[22:30]
    gsems = sc[30:34]
    ssems = sc[34:38]

    def zrow(i, _):
        zbuf[i, :] = _zeros16()
        return 0
    lax.fori_loop(0, ZR, zrow, 0)

    def stage(u, pi):
        base = ss * WPT + u
        pltpu.async_copy(src2d.at[pl.ds(base, 1)], sbufs[pi], stsems[pi])
        pltpu.async_copy(dst2d.at[pl.ds(base, 1)], dbufs[pi], stsems[pi])

    for pas in range(2):
        tabs = (h0, h1) if pas == 0 else (h2, h3)

        def zacc(q, _):
            pltpu.sync_copy(zbuf, acc.at[pl.ds(ss * RPN + q * ZR, ZR)])
            return 0
        lax.fori_loop(0, RPN // ZR, zacc, 0)
        plsc.subcore_barrier()

        def unit(u, pr, pi):
            pr1 = (pr - 1) % 4
            pi1 = (pi - 1) % 8

            @pl.when(u >= 4)
            def _():
                pltpu.make_async_copy(
                    rows[pr], acc.at[dbufs[(pi + 4) % 8].at[0]],
                    ssems[pr]).wait()

            @pl.when(u + 4 < WPT)
            def _():
                stage(u + 4, (pi + 4) % 8)

            for _ in range(2):
                pltpu.make_async_copy(dst2d.at[pl.ds(0, 1)], dbufs[pi],
                                      stsems[pi]).wait()

            @pl.when(cc == 0)
            def _():
                pltpu.async_copy(tabs[0].at[sbufs[pi].at[0]], rows[pr],
                                 gsems[pr])

            @pl.when(cc == 1)
            def _():
                pltpu.async_copy(tabs[1].at[sbufs[pi].at[0]], rows[pr],
                                 gsems[pr])

            @pl.when(u >= 1)
            def _():
                pltpu.make_async_copy(tabs[0].at[sbufs[pi1].at[0]],
                                      rows[pr1], gsems[pr1]).wait()
                pltpu.async_copy(rows[pr1], acc.at[dbufs[pi1].at[0]],
                                 ssems[pr1], add=True)

        for p in range(4):
            stage(p, p)

        def body(g, _):
            for uu in range(8):
                unit(g * 8 + uu, uu % 4, uu)
            return 0
        lax.fori_loop(0, WPT // 8, body, 0)
        pltpu.make_async_copy(tabs[0].at[sbufs[7].at[0]], rows[3],
                              gsems[3]).wait()
        pltpu.async_copy(rows[3], acc.at[dbufs[7].at[0]], ssems[3],
                         add=True)
        for p in range(4):
            pltpu.make_async_copy(rows[p], acc.at[dbufs[p].at[0]],
                                  ssems[p]).wait()
        plsc.subcore_barrier()

        pltpu.sync_copy(acc.at[pl.ds(ss * RPN, RPN)],
                        pout.at[pl.ds((pas * 2 + cc) * NP_ + ss * RPN, RPN)])
        plsc.subcore_barrier()


def _embed_body(nt, xc, xg, xp, xr, xv, d0, d1, W1, b1, W2, b2, W0,
                h0, h1, h2, h3, dinv):
    oh = (nt[...] == lax.broadcasted_iota(jnp.int32, (RB, T), 1))
    e1 = jnp.dot(oh.astype(jnp.float32), W1[...],
                 preferred_element_type=jnp.float32) + b1[...]
    w2 = W2[...]
    e2 = (xc[...] * w2[0:1, :] + xg[...] * w2[1:2, :] + xp[...] * w2[2:3, :]
          + xr[...] * w2[3:4, :] + xv[...] * w2[4:5, :]) + b2[...]
    z = jnp.concatenate([e1, e2], axis=1)
    deg = 1.0 + d0[...][:, 0:1] + d1[...][:, 0:1]
    di = lax.rsqrt(deg)
    h = jnp.dot(z, W0[...], preferred_element_type=jnp.float32) * di
    h0[...] = h[:, 0 * Q:1 * Q]
    h1[...] = h[:, 1 * Q:2 * Q]
    h2[...] = h[:, 2 * Q:3 * Q]
    h3[...] = h[:, 3 * Q:4 * Q]
    dinv[...] = di


def _embed_call(nt2, cols, d0, d1, W1, b1r, W2, b2r, W0):
    col = pl.BlockSpec((RB, 1), lambda i: (i, 0))
    dq = pl.BlockSpec((RB, Q), lambda i: (i, 0))
    full = lambda s: pl.BlockSpec(s, lambda i: (0, 0))
    qs = jax.ShapeDtypeStruct((NP_, Q), jnp.float32)
    return pl.pallas_call(
        _embed_body,
        grid=(GRID,),
        in_specs=[col, col, col, col, col, col, dq, dq,
                  full((T, H)), full((1, H)), full((5, H)), full((1, H)),
                  full((2 * H, H))],
        out_specs=[dq, dq, dq, dq, col],
        out_shape=[qs, qs, qs, qs,
                   jax.ShapeDtypeStruct((NP_, 1), jnp.float32)],
    )(nt2, *cols, d0, d1, W1, b1r, W2, b2r, W0)


def _mid_body(p0, p1, p2, p3, h0, h1, h2, h3, dinv, bb, Wn,
              o0, o1, o2, o3):
    di = dinv[...]
    z = jnp.concatenate(
        [p0[...] + h0[...], p1[...] + h1[...],
         p2[...] + h2[...], p3[...] + h3[...]], axis=1)
    z = di * z + bb[...]
    z = jnp.maximum(z, 0.0)
    h = jnp.dot(z, Wn[...], preferred_element_type=jnp.float32) * di
    o0[...] = h[:, 0 * Q:1 * Q]
    o1[...] = h[:, 1 * Q:2 * Q]
    o2[...] = h[:, 2 * Q:3 * Q]
    o3[...] = h[:, 3 * Q:4 * Q]


def _mid_call(pout, hq, dinv2, bb, Wn):
    dq = pl.BlockSpec((RB, Q), lambda i: (i, 0))
    pq = [pl.BlockSpec((RB, Q), lambda i, k=k: (k * GRID + i, 0))
          for k in range(4)]
    col = pl.BlockSpec((RB, 1), lambda i: (i, 0))
    full = lambda s: pl.BlockSpec(s, lambda i: (0, 0))
    qs = jax.ShapeDtypeStruct((NP_, Q), jnp.float32)
    return pl.pallas_call(
        _mid_body,
        grid=(GRID,),
        in_specs=[*pq, dq, dq, dq, dq, col,
                  full((1, H)), full((H, H))],
        out_specs=[dq, dq, dq, dq],
        out_shape=[qs, qs, qs, qs],
    )(pout, pout, pout, pout, *hq, dinv2, bb, Wn)


def _fin_body(p0, p1, p2, p3, h0, h1, h2, h3, dinv, bb, bt, Wo, bo,
              accum, pred):
    i = pl.program_id(0)
    di = dinv[...]
    z = jnp.concatenate(
        [p0[...] + h0[...], p1[...] + h1[...],
         p2[...] + h2[...], p3[...] + h3[...]], axis=1)
    z = di * z + bb[...]
    m = (bt[...] == lax.broadcasted_iota(jnp.int32, (RB, NG), 1))
    zaug = jnp.concatenate(
        [z, jnp.ones((RB, 1), jnp.float32), jnp.zeros((RB, 63), jnp.float32)],
        axis=1)
    contrib = lax.dot_general(m.astype(jnp.float32), zaug,
                              (((0,), (0,)), ((), ())),
                              preferred_element_type=jnp.float32)

    @pl.when(i == 0)
    def _():
        accum[...] = jnp.zeros((NG, 128), jnp.float32)

    accum[...] += contrib

    @pl.when(i == GRID - 1)
    def _():
        a = accum[...]
        pooled = a[:, :H] / jnp.maximum(a[:, H:H + 1], 1.0)
        pred[...] = jnp.dot(pooled, Wo[...],
                            preferred_element_type=jnp.float32) + bo[...]


def _fin_call(pout, hq, dinv2, bb, bt2, Wop, bop):
    dq = pl.BlockSpec((RB, Q), lambda i: (i, 0))
    pq = [pl.BlockSpec((RB, Q), lambda i, k=k: (k * GRID + i, 0))
          for k in range(4)]
    col = pl.BlockSpec((RB, 1), lambda i: (i, 0))
    full = lambda s: pl.BlockSpec(s, lambda i: (0, 0))
    acc_spec = pl.BlockSpec((NG, 128), lambda i: (0, 0))
    _, pred = pl.pallas_call(
        _fin_body,
        grid=(GRID,),
        in_specs=[*pq, dq, dq, dq, dq, col,
                  full((1, H)), col, full((H, 128)), full((1, 128))],
        out_specs=[acc_spec, acc_spec],
        out_shape=[jax.ShapeDtypeStruct((NG, 128), jnp.float32),
                   jax.ShapeDtypeStruct((NG, 128), jnp.float32)],
    )(pout, pout, pout, pout, *hq, dinv2, bb, bt2, Wop, bop)
    return pred


def kernel(node_type, c, gm, pos, r, vid, edge_index, batch,
           W1, b1, W2, b2, gcn_params, Wout, bout):
    f32, i32 = jnp.float32, jnp.int32

    src = edge_index[0].astype(i32)
    dst = edge_index[1].astype(i32)
    src2d = jnp.concatenate([src, jnp.zeros((EP - E,), i32)]).reshape(EPW, EW)
    dst2d = jnp.concatenate([dst, jnp.full((EP - E,), N, i32)]).reshape(EPW, EW)
    ones2d = jnp.ones((EW, Q), f32)

    deg0, deg1 = _deg(dst2d, ones2d)

    padc = lambda a: jnp.pad(a.astype(f32), (0, NP_ - N)).reshape(NP_, 1)
    nt2 = jnp.pad(node_type.astype(i32), (0, NP_ - N)).reshape(NP_, 1)
    cols = [padc(c), padc(gm), padc(pos), padc(r), padc(vid)]
    bt2 = jnp.pad(batch.astype(i32), (0, NP_ - N),
                  constant_values=1 << 20).reshape(NP_, 1)

    W0 = gcn_params[0][0]
    *hq, dinv2 = _embed_call(nt2, cols, deg0, deg1,
                             W1, b1.reshape(1, H), W2, b2.reshape(1, H), W0)

    for l in range(NL):
        pout = _prop(*hq, src2d, dst2d)
        bb = gcn_params[l][1].reshape(1, H)
        if l < NL - 1:
            hq = _mid_call(pout, hq, dinv2, bb, gcn_params[l + 1][0])
        else:
            Wop = jnp.pad(Wout.astype(f32), ((0, 0), (0, 128 - 4)))
            bop = jnp.pad(bout.astype(f32), (0, 128 - 4)).reshape(1, 128)
            pred = _fin_call(pout, hq, dinv2, bb, bt2, Wop, bop)

    return pred[:, :4]
